# baseline (speedup 1.0000x reference)
import functools

import numpy as np
import jax
import jax.numpy as jnp
from jax import lax
from jax.experimental import pallas as pl
from jax.experimental.pallas import tpu as pltpu
from jax.experimental.pallas import tpu_sc as plsc

_MAX_PARAMS = 524288
_LEVELS = 16
_BASE_RES = 16.0
_MAX_RES = 2048.0
_FEAT = 2
_B = 131072

_P2 = -1640531535
_P3 = 805459861


def _layout():
    log_b = np.log(_MAX_RES / _BASE_RES) / (_LEVELS - 1)
    offs, sizes, scales = [], [], []
    off = 0
    for i in range(_LEVELS):
        res = np.ceil(_BASE_RES * np.exp(i * log_b) - 1.0) + 1.0
        aligned = int((res ** 3 + 7) // 8) * 8
        sz = int(min(_MAX_PARAMS, aligned))
        offs.append(off)
        sizes.append(sz)
        scales.append(float(_BASE_RES * np.exp(i * log_b) - 1.0))
        off += sz
    return offs, sizes, scales


_OFFS, _SIZES, _SCALES = _layout()
_TOTAL = _OFFS[-1] + _SIZES[-1]

_NCOARSE = 5
_COARSE = _OFFS[_NCOARSE]
_SPLIT = 11

_NC, _NS = 2, 16
_NW = _NC * _NS
_PTS = _B // _NW


@functools.cache
def _build(lv_lo, lv_hi, s_pts, cache_coarse):
  base_row = _OFFS[lv_lo]
  end_row = _OFFS[lv_hi] if lv_hi < _LEVELS else _TOTAL
  span = end_row - base_row
  nlv = lv_hi - lv_lo
  S = s_pts
  G = S // 16
  NSUB = _PTS // S

  mesh = plsc.VectorSubcoreMesh(core_axis_name="c", subcore_axis_name="s")

  scratch = [
      pltpu.VMEM((3 * S,), jnp.float32),
      pltpu.VMEM((16 * S,), jnp.int32),
      pltpu.VMEM((16 * S,), jnp.int32),
      pltpu.VMEM((16 * S,), jnp.float32),
      pltpu.VMEM((16 * S,), jnp.float32),
      pltpu.VMEM((8, S), jnp.float32),
      pltpu.VMEM((8, S), jnp.float32),
      pltpu.VMEM((2 * nlv, S), jnp.float32),
  ]
  if cache_coarse:
    scratch.append(pltpu.VMEM_SHARED((2 * _COARSE,), jnp.float32))
  scratch += [pltpu.SemaphoreType.DMA, pltpu.SemaphoreType.DMA]

  @functools.partial(
      pl.kernel,
      out_type=jax.ShapeDtypeStruct((2 * nlv, _B), jnp.float32),
      mesh=mesh,
      compiler_params=pltpu.CompilerParams(needs_layout_passes=False),
      scratch_types=scratch,
  )
  def _hash_enc(xyz_t, table, out, xyz_v, idx0, idx1, rows0, rows1, w0, w1,
                ob, *rest):
    if cache_coarse:
      coarse, sem0, sem1 = rest
    else:
      sem0, sem1 = rest
      coarse = None
    wid = lax.axis_index("s") * _NC + lax.axis_index("c")
    tile_base = wid * _PTS

    if cache_coarse:
      sid = lax.axis_index("s")
      _CH = 8072
      _NCH = _COARSE // _CH
      for f in range(2):
        for j in range((_NCH + _NS - 1) // _NS):
          ci = sid + _NS * j

          @pl.when(ci < _NCH)
          def _():
            pltpu.sync_copy(table.at[pl.ds(f * span + ci * _CH, _CH)],
                            rows0.at[pl.ds(0, _CH)])
            pltpu.sync_copy(rows0.at[pl.ds(0, _CH)],
                            coarse.at[pl.ds(f * _COARSE + ci * _CH, _CH)])

      plsc.subcore_barrier()

    idxb = (idx0, idx1)
    rowsb = (rows0, rows1)
    wb = (w0, w1)
    sems = (sem0, sem1)

    iota = lax.iota(jnp.int32, 16)
    zero_i = jnp.zeros((16,), jnp.int32)

    def umod(h, size):
      if size & (size - 1) == 0:
        return h & (size - 1)
      lo = h & 0x7FFFFFFF
      r = lax.rem(lo, jnp.full((16,), size, jnp.int32))
      c1 = (1 << 31) % size
      r = r + jnp.where(h < 0, jnp.full((16,), c1, jnp.int32), zero_i)
      return lax.rem(r, jnp.full((16,), size, jnp.int32))

    def compute_group(lvl, g, idx_r, w_r):
      scale = _SCALES[lvl]
      size = _SIZES[lvl]
      in_spmem = cache_coarse and lvl < _NCOARSE
      off = _OFFS[lvl] if in_spmem else _OFFS[lvl] - base_row
      base16 = g * 16
      px = xyz_v[pl.ds(base16, 16)] * scale + 0.5
      py = xyz_v[pl.ds(S + base16, 16)] * scale + 0.5
      pz = xyz_v[pl.ds(2 * S + base16, 16)] * scale + 0.5
      ix = px.astype(jnp.int32)
      iy = py.astype(jnp.int32)
      iz = pz.astype(jnp.int32)
      fx = px - ix.astype(jnp.float32)
      fy = py - iy.astype(jnp.float32)
      fz = pz - iz.astype(jnp.float32)
      hx = (ix, ix + 1)
      hy = (iy * _P2, (iy + 1) * _P2)
      hz = (iz * _P3, (iz + 1) * _P3)
      wx = (1.0 - fx, fx)
      wy = (1.0 - fy, fy)
      wz = (1.0 - fz, fz)
      f1_off = _COARSE if in_spmem else span
      for c in range(8):
        dx, dy, dz = (c >> 2) & 1, (c >> 1) & 1, c & 1
        h = hx[dx] ^ hy[dy] ^ hz[dz]
        idx = umod(h, size) + off
        idx_r[pl.ds(c * S + base16, 16)] = idx
        idx_r[pl.ds(8 * S + c * S + base16, 16)] = idx + f1_off
        w_r[c, pl.ds(base16, 16)] = (wx[dx] * wy[dy]) * wz[dz]

    def apply_group(lvl, g, w_r, rows_r):
      base16 = g * 16
      acc0 = jnp.zeros((16,), jnp.float32)
      acc1 = jnp.zeros((16,), jnp.float32)
      for c in range(8):
        f0 = rows_r[pl.ds(c * S + base16, 16)]
        f1 = rows_r[pl.ds(8 * S + c * S + base16, 16)]
        w = w_r[c, pl.ds(base16, 16)]
        acc0 = acc0 + w * f0
        acc1 = acc1 + w * f1
      ch = 2 * (lvl - lv_lo)
      ob[ch, pl.ds(base16, 16)] = acc0
      ob[ch + 1, pl.ds(base16, 16)] = acc1

    def bufs_for(lvl):
      b = lvl & 1
      if cache_coarse and lvl < _NCOARSE:
        return idxb[b], rowsb[b], coarse
      return idxb[b], rowsb[b], table

    def launch_level(lvl):
      idx_r, rows_r, src = bufs_for(lvl)

      def gbody(g, carry):
        compute_group(lvl, g, idx_r, wb[lvl & 1])
        return carry

      lax.fori_loop(0, G, gbody, 0)
      return pltpu.async_copy(src.at[idx_r], rows_r, sems[lvl & 1])

    def apply_level(lvl):
      idx_r, rows_r, src = bufs_for(lvl)

      def gbody(g, carry):
        apply_group(lvl, g, wb[lvl & 1], rows_r)
        return carry

      lax.fori_loop(0, G, gbody, 0)

    def do_sub(s, carry):
      pbase = tile_base + s * S
      for d in range(3):
        pltpu.sync_copy(xyz_t.at[pl.ds(d * _B + pbase, S)],
                        xyz_v.at[pl.ds(d * S, S)])
      cp = launch_level(lv_lo)
      for lvl in range(lv_lo + 1, lv_hi):
        cp_next = launch_level(lvl)
        cp.wait()
        apply_level(lvl - 1)
        cp = cp_next
      cp.wait()
      apply_level(lv_hi - 1)
      pltpu.sync_copy(ob, out.at[:, pl.ds(pbase, S)])
      return carry

    lax.fori_loop(0, NSUB, do_sub, 0)

  return _hash_enc


def kernel(xyzs, hash_table, offsets, hash_map_sizes):
    del offsets, hash_map_sizes
    xyz_flat = xyzs.T.reshape(-1)
    r8 = _OFFS[_SPLIT]
    lo_flat = jnp.concatenate([hash_table[:r8, 0], hash_table[:r8, 1]])
    hi_flat = jnp.concatenate([hash_table[r8:, 0], hash_table[r8:, 1]])
    lo = _build(0, _SPLIT, 512, True)(xyz_flat, lo_flat)
    hi = _build(_SPLIT, _LEVELS, 1024, False)(xyz_flat, hi_flat)
    chan = jnp.concatenate([lo, hi], axis=0)
    return chan.T.reshape(_B, _LEVELS, _FEAT)


# speedup vs baseline: 1.0783x; 1.0783x over previous
import functools

import numpy as np
import jax
import jax.numpy as jnp
from jax import lax
from jax.experimental import pallas as pl
from jax.experimental.pallas import tpu as pltpu
from jax.experimental.pallas import tpu_sc as plsc

_MAX_PARAMS = 524288
_LEVELS = 16
_BASE_RES = 16.0
_MAX_RES = 2048.0
_FEAT = 2
_B = 131072

_P2 = -1640531535
_P3 = 805459861


def _layout():
    log_b = np.log(_MAX_RES / _BASE_RES) / (_LEVELS - 1)
    offs, sizes, scales = [], [], []
    off = 0
    for i in range(_LEVELS):
        res = np.ceil(_BASE_RES * np.exp(i * log_b) - 1.0) + 1.0
        aligned = int((res ** 3 + 7) // 8) * 8
        sz = int(min(_MAX_PARAMS, aligned))
        offs.append(off)
        sizes.append(sz)
        scales.append(float(_BASE_RES * np.exp(i * log_b) - 1.0))
        off += sz
    return offs, sizes, scales


_OFFS, _SIZES, _SCALES = _layout()
_TOTAL = _OFFS[-1] + _SIZES[-1]

_NLOCAL = 2
_LOCAL = _OFFS[_NLOCAL]
_NCOARSE = 5
_COARSE = _OFFS[_NCOARSE]
_CSPAN = _COARSE - _LOCAL
_SPLIT = 8

_NC, _NS = 2, 16
_NW = _NC * _NS
_PTS = _B // _NW


@functools.cache
def _build(lv_lo, lv_hi, s_pts, cache_coarse):
  base_row = _OFFS[lv_lo]
  end_row = _OFFS[lv_hi] if lv_hi < _LEVELS else _TOTAL
  span = end_row - base_row
  nlv = lv_hi - lv_lo
  S = s_pts
  G = S // 16
  NSUB = _PTS // S

  mesh = plsc.VectorSubcoreMesh(core_axis_name="c", subcore_axis_name="s")

  scratch = [
      pltpu.VMEM((3 * S,), jnp.float32),
      pltpu.VMEM((16 * S,), jnp.int32),
      pltpu.VMEM((16 * S,), jnp.int32),
      pltpu.VMEM((16 * S,), jnp.float32),
      pltpu.VMEM((16 * S,), jnp.float32),
      pltpu.VMEM((8, S), jnp.float32),
      pltpu.VMEM((8, S), jnp.float32),
      pltpu.VMEM((2 * nlv, S), jnp.float32),
  ]
  if cache_coarse:
    scratch += [
        pltpu.VMEM_SHARED((2 * _CSPAN,), jnp.float32),
        pltpu.VMEM((2 * _LOCAL,), jnp.float32),
    ]
  scratch += [pltpu.SemaphoreType.DMA, pltpu.SemaphoreType.DMA]

  @functools.partial(
      pl.kernel,
      out_type=jax.ShapeDtypeStruct((2 * nlv, _B), jnp.float32),
      mesh=mesh,
      compiler_params=pltpu.CompilerParams(needs_layout_passes=False),
      scratch_types=scratch,
  )
  def _hash_enc(xyz_t, table, out, xyz_v, idx0, idx1, rows0, rows1, w0, w1,
                ob, *rest):
    if cache_coarse:
      coarse, tl01, sem0, sem1 = rest
    else:
      sem0, sem1 = rest
      coarse = tl01 = None
    wid = lax.axis_index("s") * _NC + lax.axis_index("c")
    tile_base = wid * _PTS

    if cache_coarse:
      for f in range(2):
        pltpu.sync_copy(table.at[pl.ds(f * span, _LOCAL)],
                        tl01.at[pl.ds(f * _LOCAL, _LOCAL)])
      sid = lax.axis_index("s")
      _CH = 7152
      _NCH = _CSPAN // _CH
      for f in range(2):
        for j in range((_NCH + _NS - 1) // _NS):
          ci = sid + _NS * j

          @pl.when(ci < _NCH)
          def _():
            pltpu.sync_copy(
                table.at[pl.ds(f * span + _LOCAL + ci * _CH, _CH)],
                rows0.at[pl.ds(0, _CH)])
            pltpu.sync_copy(rows0.at[pl.ds(0, _CH)],
                            coarse.at[pl.ds(f * _CSPAN + ci * _CH, _CH)])

      plsc.subcore_barrier()

    idxb = (idx0, idx1)
    rowsb = (rows0, rows1)
    wb = (w0, w1)
    sems = (sem0, sem1)

    iota = lax.iota(jnp.int32, 16)
    zero_i = jnp.zeros((16,), jnp.int32)

    def umod(h, size):
      if size & (size - 1) == 0:
        return h & (size - 1)
      lo = h & 0x7FFFFFFF
      r = lax.rem(lo, jnp.full((16,), size, jnp.int32))
      c1 = (1 << 31) % size
      r = r + jnp.where(h < 0, jnp.full((16,), c1, jnp.int32), zero_i)
      return lax.rem(r, jnp.full((16,), size, jnp.int32))

    def corner_geometry(lvl, g):
      scale = _SCALES[lvl]
      base16 = g * 16
      px = xyz_v[pl.ds(base16, 16)] * scale + 0.5
      py = xyz_v[pl.ds(S + base16, 16)] * scale + 0.5
      pz = xyz_v[pl.ds(2 * S + base16, 16)] * scale + 0.5
      ix = px.astype(jnp.int32)
      iy = py.astype(jnp.int32)
      iz = pz.astype(jnp.int32)
      fx = px - ix.astype(jnp.float32)
      fy = py - iy.astype(jnp.float32)
      fz = pz - iz.astype(jnp.float32)
      hx = (ix, ix + 1)
      hy = (iy * _P2, (iy + 1) * _P2)
      hz = (iz * _P3, (iz + 1) * _P3)
      wx = (1.0 - fx, fx)
      wy = (1.0 - fy, fy)
      wz = (1.0 - fz, fz)
      return hx, hy, hz, wx, wy, wz

    def compute_group(lvl, g, idx_r, w_r):
      size = _SIZES[lvl]
      in_spmem = cache_coarse and lvl < _NCOARSE
      off = _OFFS[lvl] - _LOCAL if in_spmem else _OFFS[lvl] - base_row
      f1_off = _CSPAN if in_spmem else span
      base16 = g * 16
      hx, hy, hz, wx, wy, wz = corner_geometry(lvl, g)
      for c in range(8):
        dx, dy, dz = (c >> 2) & 1, (c >> 1) & 1, c & 1
        h = hx[dx] ^ hy[dy] ^ hz[dz]
        idx = umod(h, size) + off
        idx_r[pl.ds(c * S + base16, 16)] = idx
        idx_r[pl.ds(8 * S + c * S + base16, 16)] = idx + f1_off
        w_r[c, pl.ds(base16, 16)] = (wx[dx] * wy[dy]) * wz[dz]

    def local_group(lvl, g):
      size = _SIZES[lvl]
      base16 = g * 16
      hx, hy, hz, wx, wy, wz = corner_geometry(lvl, g)
      acc0 = jnp.zeros((16,), jnp.float32)
      acc1 = jnp.zeros((16,), jnp.float32)
      for c in range(8):
        dx, dy, dz = (c >> 2) & 1, (c >> 1) & 1, c & 1
        h = hx[dx] ^ hy[dy] ^ hz[dz]
        idx = umod(h, size) + _OFFS[lvl]
        f0 = plsc.load_gather(tl01, [idx])
        f1 = plsc.load_gather(tl01, [idx + _LOCAL])
        w = (wx[dx] * wy[dy]) * wz[dz]
        acc0 = acc0 + w * f0
        acc1 = acc1 + w * f1
      ch = 2 * (lvl - lv_lo)
      ob[ch, pl.ds(base16, 16)] = acc0
      ob[ch + 1, pl.ds(base16, 16)] = acc1

    def local_level(lvl):
      def gbody(g, carry):
        local_group(lvl, g)
        return carry

      lax.fori_loop(0, G, gbody, 0)

    def apply_group(lvl, g, w_r, rows_r):
      base16 = g * 16
      acc0 = jnp.zeros((16,), jnp.float32)
      acc1 = jnp.zeros((16,), jnp.float32)
      for c in range(8):
        f0 = rows_r[pl.ds(c * S + base16, 16)]
        f1 = rows_r[pl.ds(8 * S + c * S + base16, 16)]
        w = w_r[c, pl.ds(base16, 16)]
        acc0 = acc0 + w * f0
        acc1 = acc1 + w * f1
      ch = 2 * (lvl - lv_lo)
      ob[ch, pl.ds(base16, 16)] = acc0
      ob[ch + 1, pl.ds(base16, 16)] = acc1

    def bufs_for(lvl):
      b = lvl & 1
      if cache_coarse and lvl < _NCOARSE:
        return idxb[b], rowsb[b], coarse
      return idxb[b], rowsb[b], table

    first_lv = _NLOCAL if cache_coarse else lv_lo

    def launch_level(lvl):
      idx_r, rows_r, src = bufs_for(lvl)

      def gbody(g, carry):
        compute_group(lvl, g, idx_r, wb[lvl & 1])
        return carry

      lax.fori_loop(0, G, gbody, 0)
      return pltpu.async_copy(src.at[idx_r], rows_r, sems[lvl & 1])

    def apply_level(lvl):
      idx_r, rows_r, src = bufs_for(lvl)

      def gbody(g, carry):
        apply_group(lvl, g, wb[lvl & 1], rows_r)
        return carry

      lax.fori_loop(0, G, gbody, 0)

    def do_sub(s, carry):
      pbase = tile_base + s * S
      for d in range(3):
        pltpu.sync_copy(xyz_t.at[pl.ds(d * _B + pbase, S)],
                        xyz_v.at[pl.ds(d * S, S)])
      cp = launch_level(first_lv)
      if cache_coarse:
        for lvl in range(lv_lo, _NLOCAL):
          local_level(lvl)
      for lvl in range(first_lv + 1, lv_hi):
        cp_next = launch_level(lvl)
        cp.wait()
        apply_level(lvl - 1)
        cp = cp_next
      cp.wait()
      apply_level(lv_hi - 1)
      pltpu.sync_copy(ob, out.at[:, pl.ds(pbase, S)])
      return carry

    lax.fori_loop(0, NSUB, do_sub, 0)

  return _hash_enc


def kernel(xyzs, hash_table, offsets, hash_map_sizes):
    del offsets, hash_map_sizes
    xyz_flat = xyzs.T.reshape(-1)
    r8 = _OFFS[_SPLIT]
    lo_flat = jnp.concatenate([hash_table[:r8, 0], hash_table[:r8, 1]])
    hi_flat = jnp.concatenate([hash_table[r8:, 0], hash_table[r8:, 1]])
    lo = _build(0, _SPLIT, 512, True)(xyz_flat, lo_flat)
    hi = _build(_SPLIT, _LEVELS, 1024, False)(xyz_flat, hi_flat)
    chan = jnp.concatenate([lo, hi], axis=0)
    return chan.T.reshape(_B, _LEVELS, _FEAT)
